# Initial kernel scaffold; baseline (speedup 1.0000x reference)
import jax
import jax.numpy as jnp
from jax.experimental import pallas as pl


def kernel(input):
    raise NotImplementedError("write your pallas kernel here")


# speedup vs baseline: 1.6613x; 1.6613x over previous
import functools

import jax
import jax.numpy as jnp
from jax.experimental import pallas as pl
from jax.experimental.pallas import tpu as pltpu

_BH = 256


def _grad_box_kernel(x_ref, halo_ref, a_ref, out_ref, *, bh, h_blks, nc):
    h = pl.program_id(1)
    c = pl.program_id(2)
    xb = x_ref[0, 0]

    lw = jnp.concatenate([xb[:, -1:], xb[:, :-1]], axis=1)
    lanes = jax.lax.broadcasted_iota(jnp.int32, xb.shape, 1)
    lw = jnp.where(lanes == 0, 0.0, lw)

    up = jnp.concatenate([xb[-1:, :], xb[:-1, :]], axis=0)
    prev = halo_ref[0, 0, 7:8, :] * jnp.where(h == 0, 0.0, 1.0)
    rows = jax.lax.broadcasted_iota(jnp.int32, xb.shape, 0)
    up = jnp.where(rows == 0, jnp.broadcast_to(prev, xb.shape), up)

    dw = lw - xb
    dh = up - xb
    f = jnp.sqrt(dw * dw + dh * dh)

    row0 = pl.multiple_of(h * bh, bh)

    @pl.when(c == 0)
    def _():
        out_ref[0, pl.ds(row0, bh), :] = f

    @pl.when(c > 0)
    def _():
        out_ref[0, pl.ds(row0, bh), :] = out_ref[0, pl.ds(row0, bh), :] + f

    @pl.when((h == h_blks - 1) & (c == nc - 1))
    def _():
        ab = a_ref[...]
        g16 = out_ref[0].astype(jnp.bfloat16)
        t = jnp.dot(ab, g16, preferred_element_type=jnp.float32)
        out_ref[0] = jnp.dot(t.astype(jnp.bfloat16), ab,
                             preferred_element_type=jnp.float32)


def kernel(input) -> jnp.ndarray:
    x = input
    b, nc, hdim, wdim = x.shape
    r = wdim // 40
    bh = _BH
    h_blks = hdim // bh

    idx = jnp.arange(hdim)
    band = (jnp.abs(idx[:, None] - idx[None, :]) <= r).astype(jnp.bfloat16)

    out = pl.pallas_call(
        functools.partial(_grad_box_kernel, bh=bh, h_blks=h_blks, nc=nc),
        grid=(b, h_blks, nc),
        in_specs=[
            pl.BlockSpec((1, 1, bh, wdim), lambda bi, hi, ci: (bi, ci, hi, 0)),
            pl.BlockSpec(
                (1, 1, 8, wdim),
                lambda bi, hi, ci: (bi, ci, jnp.maximum(hi * (bh // 8) - 1, 0), 0),
            ),
            pl.BlockSpec((hdim, hdim), lambda bi, hi, ci: (0, 0)),
        ],
        out_specs=pl.BlockSpec((1, hdim, wdim), lambda bi, hi, ci: (bi, 0, 0)),
        out_shape=jax.ShapeDtypeStruct((b, hdim, wdim), jnp.float32),
        compiler_params=pltpu.CompilerParams(
            dimension_semantics=("parallel", "arbitrary", "arbitrary"),
            vmem_limit_bytes=100 * 1024 * 1024,
        ),
    )(x, x, band)
    return out
